# baseline (speedup 1.0000x reference)
import functools

import jax
import jax.numpy as jnp
from jax.experimental import pallas as pl
from jax.experimental.pallas import tpu as pltpu

NUM_EXPERTS = 8
EXPERT_CAPACITY = 512
BLOCK_S = 256


def _router_block(x_ref, w_ref, idx_ref, prob_ref, logits_ref, counts_ref):
    s_blk = pl.program_id(1)

    @pl.when(s_blk == 0)
    def _():
        counts_ref[...] = jnp.zeros_like(counts_ref)

    x = x_ref[0]
    w = w_ref[...]
    logits = jnp.dot(x, w, preferred_element_type=jnp.float32)

    m = jnp.max(logits, axis=-1, keepdims=True)
    ssum = jnp.sum(jnp.exp(logits - m), axis=-1, keepdims=True)
    prob_ref[0] = 1.0 / ssum

    e_iota = jax.lax.broadcasted_iota(jnp.int32, logits.shape, 1)
    idx = jnp.min(jnp.where(logits == m, e_iota, NUM_EXPERTS), axis=-1,
                  keepdims=True)
    one_hot = (e_iota == idx).astype(jnp.float32)

    r_iota = jax.lax.broadcasted_iota(jnp.int32, (BLOCK_S, BLOCK_S), 0)
    c_iota = jax.lax.broadcasted_iota(jnp.int32, (BLOCK_S, BLOCK_S), 1)
    tril = (c_iota <= r_iota).astype(jnp.float32)
    prio = jnp.dot(tril, one_hot, preferred_element_type=jnp.float32)
    prio = prio + counts_ref[...]

    counts_ref[...] = counts_ref[...] + jnp.sum(one_hot, axis=0, keepdims=True)

    keep = prio <= EXPERT_CAPACITY
    idx_ref[0] = jnp.where(keep, one_hot.astype(jnp.int32), 0)
    logits_ref[0] = logits


@jax.jit
def kernel(x, W):
    B, S, D = x.shape
    E = W.shape[1]
    grid = (B, S // BLOCK_S)
    out_types = (
        jax.ShapeDtypeStruct((B, S, E), jnp.int32),
        jax.ShapeDtypeStruct((B, S, 1), jnp.float32),
        jax.ShapeDtypeStruct((B, S, E), jnp.float32),
    )
    return pl.pallas_call(
        _router_block,
        grid=grid,
        in_specs=[
            pl.BlockSpec((1, BLOCK_S, D), lambda b, s: (b, s, 0)),
            pl.BlockSpec((D, E), lambda b, s: (0, 0)),
        ],
        out_specs=(
            pl.BlockSpec((1, BLOCK_S, E), lambda b, s: (b, s, 0)),
            pl.BlockSpec((1, BLOCK_S, 1), lambda b, s: (b, s, 0)),
            pl.BlockSpec((1, BLOCK_S, E), lambda b, s: (b, s, 0)),
        ),
        out_shape=out_types,
        scratch_shapes=[pltpu.VMEM((1, E), jnp.float32)],
        compiler_params=pltpu.CompilerParams(
            dimension_semantics=("parallel", "arbitrary"),
        ),
    )(x, W)


# speedup vs baseline: 1.2981x; 1.2981x over previous
import functools

import jax
import jax.numpy as jnp
from jax.experimental import pallas as pl
from jax.experimental.pallas import tpu as pltpu

NUM_EXPERTS = 8
EXPERT_CAPACITY = 512
BLOCK_S = 1024


def _router_block(x_ref, w_ref, idx_ref, prob_ref, logits_ref, counts_ref):
    s_blk = pl.program_id(1)

    @pl.when(s_blk == 0)
    def _():
        counts_ref[...] = jnp.zeros_like(counts_ref)

    x = x_ref[0]
    w = w_ref[...]
    logits = jnp.dot(x, w, preferred_element_type=jnp.float32)

    m = jnp.max(logits, axis=-1, keepdims=True)
    ssum = jnp.sum(jnp.exp(logits - m), axis=-1, keepdims=True)
    prob_ref[0] = 1.0 / ssum

    e_iota = jax.lax.broadcasted_iota(jnp.int32, logits.shape, 1)
    idx = jnp.min(jnp.where(logits == m, e_iota, NUM_EXPERTS), axis=-1,
                  keepdims=True)
    one_hot = (e_iota == idx).astype(jnp.float32)

    r_iota = jax.lax.broadcasted_iota(jnp.int32, (BLOCK_S, BLOCK_S), 0)
    c_iota = jax.lax.broadcasted_iota(jnp.int32, (BLOCK_S, BLOCK_S), 1)
    tril = (c_iota <= r_iota).astype(jnp.float32)
    prio = jnp.dot(tril, one_hot, preferred_element_type=jnp.float32)
    prio = prio + counts_ref[...]

    counts_ref[...] = counts_ref[...] + jnp.sum(one_hot, axis=0, keepdims=True)

    keep = prio <= EXPERT_CAPACITY
    idx_ref[0] = jnp.where(keep, one_hot.astype(jnp.int32), 0)
    logits_ref[0] = logits


@jax.jit
def kernel(x, W):
    B, S, D = x.shape
    E = W.shape[1]
    grid = (B, S // BLOCK_S)
    out_types = (
        jax.ShapeDtypeStruct((B, S, E), jnp.int32),
        jax.ShapeDtypeStruct((B, S, 1), jnp.float32),
        jax.ShapeDtypeStruct((B, S, E), jnp.float32),
    )
    return pl.pallas_call(
        _router_block,
        grid=grid,
        in_specs=[
            pl.BlockSpec((1, BLOCK_S, D), lambda b, s: (b, s, 0)),
            pl.BlockSpec((D, E), lambda b, s: (0, 0)),
        ],
        out_specs=(
            pl.BlockSpec((1, BLOCK_S, E), lambda b, s: (b, s, 0)),
            pl.BlockSpec((1, BLOCK_S, 1), lambda b, s: (b, s, 0)),
            pl.BlockSpec((1, BLOCK_S, E), lambda b, s: (b, s, 0)),
        ),
        out_shape=out_types,
        scratch_shapes=[pltpu.VMEM((1, E), jnp.float32)],
        compiler_params=pltpu.CompilerParams(
            dimension_semantics=("parallel", "arbitrary"),
        ),
    )(x, W)


# speedup vs baseline: 1.4133x; 1.0887x over previous
import jax
import jax.numpy as jnp
from jax.experimental import pallas as pl
from jax.experimental.pallas import tpu as pltpu

NUM_EXPERTS = 8
EXPERT_CAPACITY = 512
BLOCK_S = 1024
CHUNK = 128


def _router_block(x_ref, w_ref, idx_ref, prob_ref, logits_ref, counts_ref):
    s_blk = pl.program_id(1)

    @pl.when(s_blk == 0)
    def _():
        counts_ref[...] = jnp.zeros_like(counts_ref)

    x = x_ref[0]
    w = w_ref[...]
    logits = jnp.dot(x, w, preferred_element_type=jnp.float32)

    m = jnp.max(logits, axis=-1, keepdims=True)
    ssum = jnp.sum(jnp.exp(logits - m), axis=-1, keepdims=True)
    prob_ref[0] = 1.0 / ssum

    e_iota = jax.lax.broadcasted_iota(jnp.int32, logits.shape, 1)
    idx = jnp.min(jnp.where(logits == m, e_iota, NUM_EXPERTS), axis=-1,
                  keepdims=True)
    one_hot = (e_iota == idx).astype(jnp.float32)

    nc = BLOCK_S // CHUNK
    oh_c = one_hot.reshape(nc, CHUNK, NUM_EXPERTS)
    r_iota = jax.lax.broadcasted_iota(jnp.int32, (CHUNK, CHUNK), 0)
    c_iota = jax.lax.broadcasted_iota(jnp.int32, (CHUNK, CHUNK), 1)
    tril = (c_iota <= r_iota).astype(jnp.float32)
    prio_c = jax.lax.dot_general(
        jnp.broadcast_to(tril, (nc, CHUNK, CHUNK)), oh_c,
        (((2,), (1,)), ((0,), (0,))),
        preferred_element_type=jnp.float32)
    chunk_tot = jnp.sum(oh_c, axis=1)
    cr = jax.lax.broadcasted_iota(jnp.int32, (nc, nc), 0)
    cc = jax.lax.broadcasted_iota(jnp.int32, (nc, nc), 1)
    excl = (cc < cr).astype(jnp.float32)
    chunk_off = jnp.dot(excl, chunk_tot,
                        preferred_element_type=jnp.float32)
    prio = (prio_c + chunk_off[:, None, :]).reshape(BLOCK_S, NUM_EXPERTS)
    prio = prio + counts_ref[...]

    counts_ref[...] = counts_ref[...] + jnp.sum(chunk_tot, axis=0,
                                                keepdims=True)

    keep = prio <= EXPERT_CAPACITY
    idx_ref[0] = jnp.where(keep, one_hot.astype(jnp.int32), 0)
    logits_ref[0] = logits


@jax.jit
def kernel(x, W):
    B, S, D = x.shape
    E = W.shape[1]
    grid = (B, S // BLOCK_S)
    out_types = (
        jax.ShapeDtypeStruct((B, S, E), jnp.int32),
        jax.ShapeDtypeStruct((B, S, 1), jnp.float32),
        jax.ShapeDtypeStruct((B, S, E), jnp.float32),
    )
    return pl.pallas_call(
        _router_block,
        grid=grid,
        in_specs=[
            pl.BlockSpec((1, BLOCK_S, D), lambda b, s: (b, s, 0)),
            pl.BlockSpec((D, E), lambda b, s: (0, 0)),
        ],
        out_specs=(
            pl.BlockSpec((1, BLOCK_S, E), lambda b, s: (b, s, 0)),
            pl.BlockSpec((1, BLOCK_S, 1), lambda b, s: (b, s, 0)),
            pl.BlockSpec((1, BLOCK_S, E), lambda b, s: (b, s, 0)),
        ),
        out_shape=out_types,
        scratch_shapes=[pltpu.VMEM((1, E), jnp.float32)],
        compiler_params=pltpu.CompilerParams(
            dimension_semantics=("parallel", "arbitrary"),
        ),
    )(x, W)
